# Initial kernel scaffold; baseline (speedup 1.0000x reference)
import jax
import jax.numpy as jnp
from jax.experimental import pallas as pl


def kernel(pred, score, target):
    raise NotImplementedError("write your pallas kernel here")


# speedup vs baseline: 14.4203x; 14.4203x over previous
import functools

import jax
import jax.numpy as jnp
from jax.experimental import pallas as pl
from jax.experimental.pallas import tpu as pltpu

NHI = 32
NLO = 128
NBINS = NHI * NLO


def _pick_chunk(P):
    for ch in (9216, 4608, 2304, 1152, 768, 512, 384, 256, 128):
        if P % ch == 0:
            return ch
    return P


def _lovasz_kernel(inv_ref, x_ref, t_ref, out_ref, hist_ref, *, C, CH,
                   nsteps, nstep_inner):
    b = pl.program_id(0)
    i = pl.program_id(1)
    step = b * nstep_inner + i

    @pl.when(step == 0)
    def _():
        hist_ref[...] = jnp.zeros_like(hist_ref)

    x = x_ref[0]
    t = t_ref[0]

    best = t[0:1]
    lab = jnp.zeros((1, CH), jnp.int32)
    for c in range(1, C):
        row = t[c:c + 1]
        m = row > best
        best = jnp.where(m, row, best)
        lab = jnp.where(m, c, lab)

    inv = inv_ref[0, 0]
    iota_hi = jax.lax.broadcasted_iota(jnp.int32, (NHI, CH), 0)
    iota_lo = jax.lax.broadcasted_iota(jnp.int32, (NLO, CH), 0)

    for c in range(C):
        fgf = (lab == c).astype(jnp.float32)
        e = jnp.abs(fgf - x[c:c + 1])
        idx = (NBINS * (1.0 - e * inv)).astype(jnp.int32)
        idx = jnp.clip(idx, 0, NBINS - 1)
        hi = idx // NLO
        lo = idx - hi * NLO
        ohi = (hi == iota_hi).astype(jnp.bfloat16)
        olo = (lo == iota_lo).astype(jnp.bfloat16)
        e_hi = e.astype(jnp.bfloat16)
        e_lo = (e - e_hi.astype(jnp.float32)).astype(jnp.bfloat16)
        fgb = fgf.astype(jnp.bfloat16)
        A = jnp.concatenate(
            [ohi, ohi * fgb, ohi * e_hi, ohi * e_lo], axis=0)
        upd = jax.lax.dot_general(
            A, olo, (((1,), (1,)), ((), ())),
            preferred_element_type=jnp.float32)
        hist_ref[c] = hist_ref[c] + upd

    @pl.when(step == nsteps - 1)
    def _():
        iu = jax.lax.broadcasted_iota(jnp.int32, (NLO, NLO), 0)
        ju = jax.lax.broadcasted_iota(jnp.int32, (NLO, NLO), 1)
        U = (iu <= ju).astype(jnp.float32)
        O = jnp.ones((NLO, NLO), jnp.float32)
        il = jax.lax.broadcasted_iota(jnp.int32, (NHI, NHI), 0)
        jl = jax.lax.broadcasted_iota(jnp.int32, (NHI, NHI), 1)
        L = (jl < il).astype(jnp.float32)

        def dn(a, bm):
            return jax.lax.dot_general(a, bm, (((1,), (0,)), ((), ())),
                                       preferred_element_type=jnp.float32)

        total = jnp.float32(0.0)
        for c in range(C):
            h = hist_ref[c]
            n = h[0:NHI]
            g = h[NHI:2 * NHI]
            es = h[2 * NHI:3 * NHI] + h[3 * NHI:4 * NHI]
            R = dn(n, U) + dn(L, dn(n, O))
            G = dn(g, U) + dn(L, dn(g, O))
            gts = jnp.sum(g)
            un_in = jnp.maximum(gts + R - G, 1.0)
            j_in = jnp.where(R > 0.5, 1.0 - (gts - G) / un_in, 0.0)
            Rx = R - n
            Gx = G - g
            un_ex = jnp.maximum(gts + Rx - Gx, 1.0)
            j_ex = jnp.where(Rx > 0.5, 1.0 - (gts - Gx) / un_ex, 0.0)
            ebar = es / jnp.maximum(n, 1.0)
            total = total + jnp.sum(ebar * (j_in - j_ex))
        out_ref[...] = jnp.broadcast_to(total / C, (1, 1))


def _lovasz_pallas(x, t, interpret=False):
    Bq, C, P = x.shape
    CH = _pick_chunk(P)
    nstep_inner = P // CH
    nsteps = Bq * nstep_inner
    inv = (1.0 / (jnp.max(jnp.abs(x)) + 1.0)).reshape(1, 1)
    out = pl.pallas_call(
        functools.partial(_lovasz_kernel, C=C, CH=CH, nsteps=nsteps,
                          nstep_inner=nstep_inner),
        grid=(Bq, nstep_inner),
        in_specs=[
            pl.BlockSpec(memory_space=pltpu.SMEM),
            pl.BlockSpec((1, C, CH), lambda b, i: (b, 0, i)),
            pl.BlockSpec((1, C, CH), lambda b, i: (b, 0, i)),
        ],
        out_specs=pl.BlockSpec((1, 1), lambda b, i: (0, 0)),
        out_shape=jax.ShapeDtypeStruct((1, 1), jnp.float32),
        scratch_shapes=[pltpu.VMEM((C, 4 * NHI, NLO), jnp.float32)],
        interpret=interpret,
    )(inv, x, t)
    return out[0, 0]


def kernel(pred, score, target):
    del score
    Bq, C = pred.shape[1], pred.shape[2]
    P = pred.shape[3] * pred.shape[4]
    x = pred.reshape(Bq, C, P).astype(jnp.float32)
    t = target.reshape(Bq, C, P).astype(jnp.float32)
    return _lovasz_pallas(x, t)


# speedup vs baseline: 20.1384x; 1.3965x over previous
import functools

import jax
import jax.numpy as jnp
from jax.experimental import pallas as pl
from jax.experimental.pallas import tpu as pltpu

NHI = 32
NLO = 128
NBINS = NHI * NLO


def _pick_chunk(P):
    for ch in (9216, 4608, 2304, 1152, 768, 512, 384, 256, 128):
        if P % ch == 0:
            return ch
    return P


def _hist_kernel(inv_ref, x_ref, t_ref, hist_ref, *, C, CH):
    i = pl.program_id(1)

    @pl.when(i == 0)
    def _():
        hist_ref[...] = jnp.zeros_like(hist_ref)

    x = x_ref[0]
    t = t_ref[0]

    best = t[0:1]
    lab = jnp.zeros((1, CH), jnp.int32)
    for c in range(1, C):
        row = t[c:c + 1]
        m = row > best
        best = jnp.where(m, row, best)
        lab = jnp.where(m, c, lab)

    inv = inv_ref[0, 0]
    iota_c = jax.lax.broadcasted_iota(jnp.int32, (C, CH), 0)
    fg_all = (lab == iota_c)
    fgf_all = fg_all.astype(jnp.float32)
    e_all = jnp.abs(fgf_all - x)
    idx = (NBINS * (1.0 - e_all * inv)).astype(jnp.int32)
    idx = jnp.clip(idx, 0, NBINS - 1)
    hi_all = jnp.right_shift(idx, 7)
    lo_all = jnp.bitwise_and(idx, NLO - 1)
    eb_all = e_all.astype(jnp.bfloat16)
    fgb_all = fg_all.astype(jnp.bfloat16)

    iota_hi = jax.lax.broadcasted_iota(jnp.int32, (NHI, CH), 0)
    iota_lo = jax.lax.broadcasted_iota(jnp.int32, (NLO, CH), 0)

    for c in range(C):
        ohi = (hi_all[c:c + 1] == iota_hi).astype(jnp.bfloat16)
        olo = (lo_all[c:c + 1] == iota_lo).astype(jnp.bfloat16)
        A = jnp.concatenate(
            [ohi, ohi * fgb_all[c:c + 1], ohi * eb_all[c:c + 1]],
            axis=0)
        upd = jax.lax.dot_general(
            A, olo, (((1,), (1,)), ((), ())),
            preferred_element_type=jnp.float32)
        hist_ref[0, c] = hist_ref[0, c] + upd


def _reduce_kernel(hist_ref, out_ref, *, C, NB):
    iu = jax.lax.broadcasted_iota(jnp.int32, (NLO, NLO), 0)
    ju = jax.lax.broadcasted_iota(jnp.int32, (NLO, NLO), 1)
    U = (iu <= ju).astype(jnp.float32)
    O = jnp.ones((NLO, NLO), jnp.float32)
    il = jax.lax.broadcasted_iota(jnp.int32, (NHI, NHI), 0)
    jl = jax.lax.broadcasted_iota(jnp.int32, (NHI, NHI), 1)
    L = (jl < il).astype(jnp.float32)

    def dn(a, bm):
        return jax.lax.dot_general(a, bm, (((1,), (0,)), ((), ())),
                                   preferred_element_type=jnp.float32)

    total = jnp.float32(0.0)
    for c in range(C):
        h = hist_ref[0, c]
        for b in range(1, NB):
            h = h + hist_ref[b, c]
        n = h[0:NHI]
        g = h[NHI:2 * NHI]
        es = h[2 * NHI:3 * NHI]
        R = dn(n, U) + dn(L, dn(n, O))
        G = dn(g, U) + dn(L, dn(g, O))
        gts = jnp.sum(g)
        un_in = jnp.maximum(gts + R - G, 1.0)
        j_in = jnp.where(R > 0.5, 1.0 - (gts - G) / un_in, 0.0)
        Rx = R - n
        Gx = G - g
        un_ex = jnp.maximum(gts + Rx - Gx, 1.0)
        j_ex = jnp.where(Rx > 0.5, 1.0 - (gts - Gx) / un_ex, 0.0)
        ebar = es / jnp.maximum(n, 1.0)
        total = total + jnp.sum(ebar * (j_in - j_ex))
    out_ref[...] = jnp.broadcast_to(total / C, (1, 1))


def _lovasz_pallas(x, t, interpret=False):
    Bq, C, P = x.shape
    CH = _pick_chunk(P)
    nstep_inner = P // CH
    inv = (1.0 / (jnp.max(jnp.abs(x)) + 1.0)).reshape(1, 1)
    hist = pl.pallas_call(
        functools.partial(_hist_kernel, C=C, CH=CH),
        grid=(Bq, nstep_inner),
        in_specs=[
            pl.BlockSpec(memory_space=pltpu.SMEM),
            pl.BlockSpec((1, C, CH), lambda b, i: (b, 0, i)),
            pl.BlockSpec((1, C, CH), lambda b, i: (b, 0, i)),
        ],
        out_specs=pl.BlockSpec((1, C, 3 * NHI, NLO), lambda b, i: (b, 0, 0, 0)),
        out_shape=jax.ShapeDtypeStruct((Bq, C, 3 * NHI, NLO), jnp.float32),
        compiler_params=pltpu.CompilerParams(
            dimension_semantics=("parallel", "arbitrary")),
        interpret=interpret,
    )(inv, x, t)
    out = pl.pallas_call(
        functools.partial(_reduce_kernel, C=C, NB=Bq),
        out_shape=jax.ShapeDtypeStruct((1, 1), jnp.float32),
        interpret=interpret,
    )(hist)
    return out[0, 0]


def kernel(pred, score, target):
    del score
    Bq, C = pred.shape[1], pred.shape[2]
    P = pred.shape[3] * pred.shape[4]
    x = pred.reshape(Bq, C, P).astype(jnp.float32)
    t = target.reshape(Bq, C, P).astype(jnp.float32)
    return _lovasz_pallas(x, t)


# speedup vs baseline: 26.0893x; 1.2955x over previous
import functools

import jax
import jax.numpy as jnp
from jax.experimental import pallas as pl
from jax.experimental.pallas import tpu as pltpu

NHI = 32
NLO = 64
NBINS = NHI * NLO
_LOG2_NLO = 6


def _pick_chunk(P):
    for ch in (9216, 4608, 2304, 1152, 768, 512, 384, 256, 128):
        if P % ch == 0:
            return ch
    return P


def _hist_kernel(inv_ref, x_ref, t_ref, hist_ref, *, C, CH):
    i = pl.program_id(1)

    @pl.when(i == 0)
    def _():
        hist_ref[...] = jnp.zeros_like(hist_ref)

    x = x_ref[0]
    t = t_ref[0]

    iota_c = jax.lax.broadcasted_iota(jnp.int32, (C, CH), 0)
    mx = jnp.max(t, axis=0, keepdims=True)
    lab = jnp.min(jnp.where(t == mx, iota_c, C), axis=0, keepdims=True)

    inv = inv_ref[0, 0]
    fg_all = (lab == iota_c)
    fgf_all = fg_all.astype(jnp.float32)
    e_all = jnp.abs(fgf_all - x)
    idx = (NBINS * (1.0 - e_all * inv)).astype(jnp.int32)
    idx = jnp.clip(idx, 0, NBINS - 1)
    hi_all = jnp.right_shift(idx, _LOG2_NLO)
    lo_all = jnp.bitwise_and(idx, NLO - 1)
    eb_all = e_all.astype(jnp.bfloat16)
    fgb_all = fg_all.astype(jnp.bfloat16)

    iota_hi = jax.lax.broadcasted_iota(jnp.int32, (NHI, CH), 0)
    iota_lo = jax.lax.broadcasted_iota(jnp.int32, (NLO, CH), 0)

    for c in range(C):
        ohi = (hi_all[c:c + 1] == iota_hi).astype(jnp.bfloat16)
        olo = (lo_all[c:c + 1] == iota_lo).astype(jnp.bfloat16)
        A = jnp.concatenate(
            [ohi, ohi * fgb_all[c:c + 1], ohi * eb_all[c:c + 1]],
            axis=0)
        upd = jax.lax.dot_general(
            A, olo, (((1,), (1,)), ((), ())),
            preferred_element_type=jnp.float32)
        hist_ref[0, c] = hist_ref[0, c] + upd


def _reduce_kernel(hist_ref, out_ref, *, C, NB):
    iu = jax.lax.broadcasted_iota(jnp.int32, (NLO, NLO), 0)
    ju = jax.lax.broadcasted_iota(jnp.int32, (NLO, NLO), 1)
    U = (iu <= ju).astype(jnp.float32)
    O = jnp.ones((NLO, NLO), jnp.float32)
    il = jax.lax.broadcasted_iota(jnp.int32, (NHI, NHI), 0)
    jl = jax.lax.broadcasted_iota(jnp.int32, (NHI, NHI), 1)
    L = (jl < il).astype(jnp.float32)

    def dn(a, bm):
        return jax.lax.dot_general(a, bm, (((1,), (0,)), ((), ())),
                                   preferred_element_type=jnp.float32)

    total = jnp.float32(0.0)
    for c in range(C):
        h = hist_ref[0, c]
        for b in range(1, NB):
            h = h + hist_ref[b, c]
        n = h[0:NHI]
        g = h[NHI:2 * NHI]
        es = h[2 * NHI:3 * NHI]
        R = dn(n, U) + dn(L, dn(n, O))
        G = dn(g, U) + dn(L, dn(g, O))
        gts = jnp.sum(g)
        un_in = jnp.maximum(gts + R - G, 1.0)
        j_in = jnp.where(R > 0.5, 1.0 - (gts - G) / un_in, 0.0)
        Rx = R - n
        Gx = G - g
        un_ex = jnp.maximum(gts + Rx - Gx, 1.0)
        j_ex = jnp.where(Rx > 0.5, 1.0 - (gts - Gx) / un_ex, 0.0)
        ebar = es / jnp.maximum(n, 1.0)
        total = total + jnp.sum(ebar * (j_in - j_ex))
    out_ref[...] = jnp.broadcast_to(total / C, (1, 1))


def _lovasz_pallas(x, t, interpret=False):
    Bq, C, P = x.shape
    CH = _pick_chunk(P)
    nstep_inner = P // CH
    inv = (1.0 / (jnp.max(jnp.abs(x)) + 1.0)).reshape(1, 1)
    hist = pl.pallas_call(
        functools.partial(_hist_kernel, C=C, CH=CH),
        grid=(Bq, nstep_inner),
        in_specs=[
            pl.BlockSpec(memory_space=pltpu.SMEM),
            pl.BlockSpec((1, C, CH), lambda b, i: (b, 0, i)),
            pl.BlockSpec((1, C, CH), lambda b, i: (b, 0, i)),
        ],
        out_specs=pl.BlockSpec((1, C, 3 * NHI, NLO), lambda b, i: (b, 0, 0, 0)),
        out_shape=jax.ShapeDtypeStruct((Bq, C, 3 * NHI, NLO), jnp.float32),
        interpret=interpret,
    )(inv, x, t)
    out = pl.pallas_call(
        functools.partial(_reduce_kernel, C=C, NB=Bq),
        out_shape=jax.ShapeDtypeStruct((1, 1), jnp.float32),
        interpret=interpret,
    )(hist)
    return out[0, 0]


def kernel(pred, score, target):
    del score
    Bq, C = pred.shape[1], pred.shape[2]
    P = pred.shape[3] * pred.shape[4]
    x = pred.reshape(Bq, C, P).astype(jnp.float32)
    t = target.reshape(Bq, C, P).astype(jnp.float32)
    return _lovasz_pallas(x, t)


# speedup vs baseline: 34.1498x; 1.3090x over previous
import functools

import jax
import jax.numpy as jnp
from jax.experimental import pallas as pl
from jax.experimental.pallas import tpu as pltpu

NHI = 32
NLO = 64
NBINS = NHI * NLO
_LOG2_NLO = 6


def _pick_chunk(P):
    for ch in (9216, 4608, 2304, 1152, 768, 512, 384, 256, 128):
        if P % ch == 0:
            return ch
    return P


def _hist_kernel(inv_ref, x_ref, t_ref, hist_ref, *, C, CH):
    i = pl.program_id(1)

    @pl.when(i == 0)
    def _():
        hist_ref[...] = jnp.zeros_like(hist_ref)

    x = x_ref[0]
    t = t_ref[0]

    iota_c = jax.lax.broadcasted_iota(jnp.int32, (C, CH), 0)
    mx = jnp.max(t, axis=0, keepdims=True)
    lab = jnp.min(jnp.where(t == mx, iota_c, C), axis=0, keepdims=True)

    inv = inv_ref[0, 0]
    fg_all = (lab == iota_c)
    fgf_all = fg_all.astype(jnp.float32)
    e_all = jnp.abs(fgf_all - x)
    idx = (NBINS * (1.0 - e_all * inv)).astype(jnp.int32)
    idx = jnp.clip(idx, 0, NBINS - 1)
    hi_all = jnp.right_shift(idx, _LOG2_NLO)
    lo_all = jnp.bitwise_and(idx, NLO - 1)
    fgb_all = fg_all.astype(jnp.bfloat16)

    iota_hi = jax.lax.broadcasted_iota(jnp.int32, (NHI, CH), 0)
    iota_lo = jax.lax.broadcasted_iota(jnp.int32, (NLO, CH), 0)

    for c in range(C):
        ohi = (hi_all[c:c + 1] == iota_hi).astype(jnp.bfloat16)
        olo = (lo_all[c:c + 1] == iota_lo).astype(jnp.bfloat16)
        A = jnp.concatenate(
            [ohi, ohi * fgb_all[c:c + 1]], axis=0)
        upd = jax.lax.dot_general(
            A, olo, (((1,), (1,)), ((), ())),
            preferred_element_type=jnp.float32)
        hist_ref[0, c] = hist_ref[0, c] + upd


def _reduce_kernel(inv_ref, hist_ref, out_ref, *, C, NB):
    iu = jax.lax.broadcasted_iota(jnp.int32, (NLO, NLO), 0)
    ju = jax.lax.broadcasted_iota(jnp.int32, (NLO, NLO), 1)
    U = (iu <= ju).astype(jnp.float32)
    O = jnp.ones((NLO, NLO), jnp.float32)
    il = jax.lax.broadcasted_iota(jnp.int32, (NHI, NHI), 0)
    jl = jax.lax.broadcasted_iota(jnp.int32, (NHI, NHI), 1)
    L = (jl < il).astype(jnp.float32)

    def dn(a, bm):
        return jax.lax.dot_general(a, bm, (((1,), (0,)), ((), ())),
                                   preferred_element_type=jnp.float32)

    bidx = (jax.lax.broadcasted_iota(jnp.int32, (NHI, NLO), 0) * NLO +
            jax.lax.broadcasted_iota(jnp.int32, (NHI, NLO), 1))
    centers = ((1.0 - (bidx.astype(jnp.float32) + 0.5) / NBINS) /
               inv_ref[0, 0])

    total = jnp.float32(0.0)
    for c in range(C):
        h = hist_ref[0, c]
        for b in range(1, NB):
            h = h + hist_ref[b, c]
        n = h[0:NHI]
        g = h[NHI:2 * NHI]
        R = dn(n, U) + dn(L, dn(n, O))
        G = dn(g, U) + dn(L, dn(g, O))
        gts = jnp.sum(g)
        un_in = jnp.maximum(gts + R - G, 1.0)
        j_in = jnp.where(R > 0.5, 1.0 - (gts - G) / un_in, 0.0)
        Rx = R - n
        Gx = G - g
        un_ex = jnp.maximum(gts + Rx - Gx, 1.0)
        j_ex = jnp.where(Rx > 0.5, 1.0 - (gts - Gx) / un_ex, 0.0)
        total = total + jnp.sum(centers * (j_in - j_ex))
    out_ref[...] = jnp.broadcast_to(total / C, (1, 1))


def _lovasz_pallas(x, t, interpret=False):
    Bq, C, P = x.shape
    CH = _pick_chunk(P)
    nstep_inner = P // CH
    inv = (1.0 / (jnp.max(jnp.abs(x)) + 1.0)).reshape(1, 1)
    hist = pl.pallas_call(
        functools.partial(_hist_kernel, C=C, CH=CH),
        grid=(Bq, nstep_inner),
        in_specs=[
            pl.BlockSpec(memory_space=pltpu.SMEM),
            pl.BlockSpec((1, C, CH), lambda b, i: (b, 0, i)),
            pl.BlockSpec((1, C, CH), lambda b, i: (b, 0, i)),
        ],
        out_specs=pl.BlockSpec((1, C, 2 * NHI, NLO), lambda b, i: (b, 0, 0, 0)),
        out_shape=jax.ShapeDtypeStruct((Bq, C, 2 * NHI, NLO), jnp.float32),
        interpret=interpret,
    )(inv, x, t)
    out = pl.pallas_call(
        functools.partial(_reduce_kernel, C=C, NB=Bq),
        in_specs=[
            pl.BlockSpec(memory_space=pltpu.SMEM),
            pl.BlockSpec(memory_space=pltpu.VMEM),
        ],
        out_shape=jax.ShapeDtypeStruct((1, 1), jnp.float32),
        interpret=interpret,
    )(inv, hist)
    return out[0, 0]


def kernel(pred, score, target):
    del score
    Bq, C = pred.shape[1], pred.shape[2]
    P = pred.shape[3] * pred.shape[4]
    x = pred.reshape(Bq, C, P).astype(jnp.float32)
    t = target.reshape(Bq, C, P).astype(jnp.float32)
    return _lovasz_pallas(x, t)
